# baseline (speedup 1.0000x reference)
import functools

import jax
import jax.numpy as jnp
from jax import lax
from jax.experimental import pallas as pl
from jax.experimental.pallas import tpu as pltpu
from jax.experimental.pallas import tpu_sc as plsc

_N = 10000
_DIN = 128
_DH = 64
_E = 320000
_N3 = 3 * _N
_E3 = 3 * _E
_NW = 32
_EPW = _E // _NW
_C = 800
_CPW = 13
_EPWP = _CPW * _C
_SUB = 80
_NSUB = _C // _SUB
_NTILE = 16
_STRIPE = _N // _NTILE


def _proj_tail(r, hp, asrc_ref, adst_ref, hp_ref, ssrc_ref, sdst_ref, m_ref):
    hp_ref[0] = hp
    ssrc = jnp.sum(hp * asrc_ref[0, 0][None, :], axis=1)
    sdst = jnp.sum(hp * adst_ref[0, 0][None, :], axis=1)
    ssrc_ref[0, 0] = ssrc
    sdst_ref[0, 0] = sdst
    z = jnp.max(ssrc) + jnp.max(sdst)
    lm = jnp.maximum(z, 0.2 * z)

    @pl.when(r == 0)
    def _():
        m_ref[0, 0] = lm

    @pl.when(r != 0)
    def _():
        m_ref[0, 0] = jnp.maximum(m_ref[0, 0], lm)


def _proj0_body(x_ref, w_ref, asrc_ref, adst_ref,
                hp_ref, ssrc_ref, sdst_ref, m_ref):
    r = pl.program_id(0)
    hp = jnp.dot(x_ref[...], w_ref[0], preferred_element_type=jnp.float32)
    _proj_tail(r, hp, asrc_ref, adst_ref, hp_ref, ssrc_ref, sdst_ref, m_ref)


def _finish(raw_ref, den_ref, b_ref):
    raw = raw_ref[0, 0] + raw_ref[1, 0]
    den = jnp.sum(den_ref[:, 0, 0, :], axis=0)
    return raw / (den + 1e-16)[:, None] + b_ref[0, 0][None, :]


def _finproj_body(raw_ref, den_ref, b_ref, w_ref, asrc_ref, adst_ref,
                  hp_ref, ssrc_ref, sdst_ref, m_ref):
    r = pl.program_id(0)
    h = jnp.maximum(_finish(raw_ref, den_ref, b_ref), 0.0)
    hp = jnp.dot(h, w_ref[0], preferred_element_type=jnp.float32)
    _proj_tail(r, hp, asrc_ref, adst_ref, hp_ref, ssrc_ref, sdst_ref, m_ref)


def _finlast_body(raw_ref, den_ref, b_ref, out_ref):
    out_ref[0] = _finish(raw_ref, den_ref, b_ref)


_PROJ_OUT_SHAPE = [
    jax.ShapeDtypeStruct((3, _N, _DH), jnp.float32),
    jax.ShapeDtypeStruct((3, 1, _N), jnp.float32),
    jax.ShapeDtypeStruct((3, 1, _N), jnp.float32),
    jax.ShapeDtypeStruct((1, 1), jnp.float32),
]
_PROJ_OUT_SPECS = [
    pl.BlockSpec((1, _N, _DH), lambda r: (r, 0, 0)),
    pl.BlockSpec((1, 1, _N), lambda r: (r, 0, 0)),
    pl.BlockSpec((1, 1, _N), lambda r: (r, 0, 0)),
    pl.BlockSpec(memory_space=pltpu.SMEM),
]
_W_SPEC0 = pl.BlockSpec((1, _DIN, _DH), lambda r: (r, 0, 0))
_W_SPEC = pl.BlockSpec((1, _DH, _DH), lambda r: (r, 0, 0))
_A_SPEC = pl.BlockSpec((1, 1, _DH), lambda r: (r, 0, 0))
_RAW_SPEC = pl.BlockSpec((2, 1, _N, _DH), lambda r: (0, r, 0, 0))
_DEN_SPEC = pl.BlockSpec((_NW, 1, 1, _N), lambda r: (0, r, 0, 0))


def _proj0(x, w, asrc, adst):
    return pl.pallas_call(
        _proj0_body,
        grid=(3,),
        in_specs=[
            pl.BlockSpec((_N, _DIN), lambda r: (0, 0)),
            _W_SPEC0, _A_SPEC, _A_SPEC,
        ],
        out_specs=_PROJ_OUT_SPECS,
        out_shape=_PROJ_OUT_SHAPE,
    )(x, w, asrc, adst)


def _finproj(raw, den, b, w, asrc, adst):
    return pl.pallas_call(
        _finproj_body,
        grid=(3,),
        in_specs=[_RAW_SPEC, _DEN_SPEC, _A_SPEC, _W_SPEC, _A_SPEC, _A_SPEC],
        out_specs=_PROJ_OUT_SPECS,
        out_shape=_PROJ_OUT_SHAPE,
    )(raw, den, b, w, asrc, adst)


def _finlast(raw, den, b):
    return pl.pallas_call(
        _finlast_body,
        grid=(3,),
        in_specs=[_RAW_SPEC, _DEN_SPEC, _A_SPEC],
        out_specs=pl.BlockSpec((1, _N, _DH), lambda r: (r, 0, 0)),
        out_shape=jax.ShapeDtypeStruct((3, _N, _DH), jnp.float32),
    )(raw, den, b)


_SENT = -1e30
_NPAD = _N + 16


def _edge_body(ed_h, hp_h, ssrc_h, sdst_h,
               m_h, raw_o, den_o,
               ssrc_v, sdst_v, den_v, eb0, d20,
               rows0, m_v, qbuf, out_sh, sg0, ss0):
    cid = lax.axis_index("c")
    sid = lax.axis_index("s")
    wid = cid * _NTILE + sid
    zeros16 = jnp.zeros((16,), jnp.float32)
    sent16 = jnp.full((16,), _SENT, jnp.float32)
    iota16 = lax.iota(jnp.int32, 16)
    eb = (eb0,)
    d2 = (d20,)
    rows = (rows0,)
    sg = (sg0,)
    ss = (ss0,)

    pltpu.sync_copy(ssrc_h, ssrc_v)
    pltpu.sync_copy(sdst_h, sdst_v.at[pl.ds(0, _N)])
    sdst_v[pl.ds(_N, 16)] = sent16
    pltpu.sync_copy(m_h, m_v)

    def _zden(i, carry):
        den_v[pl.ds(i * 16, 16)] = zeros16
        return carry

    lax.fori_loop(0, _NPAD // 16, _zden, 0)

    def _zrow(i, carry):
        for c4 in range(_DH // 16):
            rows0[i, pl.ds(c4 * 16, 16)] = zeros16
        return carry

    lax.fori_loop(0, _C, _zrow, 0)
    zst = _NPAD // _NTILE
    pltpu.sync_copy(rows0.at[pl.ds(0, zst)],
                    out_sh.at[pl.ds(sid * zst, zst)])
    plsc.subcore_barrier()

    mvec = m_v[...]

    def _fill_d2(b):
        for j in range(_NSUB):
            for g in range(_SUB // 16):
                d2[b][j, pl.ds(g * 16, 16)] = \
                    eb[b][1, pl.ds(j * _SUB + g * 16, 16)]

    def _start_gathers(b):
        return [pltpu.async_copy(
                    hp_h.at[eb[b].at[0, pl.ds(j * _SUB, _SUB)]],
                    rows[b].at[pl.ds(j * _SUB, _SUB)], sg[b])
                for j in range(_NSUB)]

    def _start_scatters(b):
        return [pltpu.async_copy(rows[b].at[pl.ds(j * _SUB, _SUB)],
                                 out_sh.at[d2[b].at[j]], ss[b], add=True)
                for j in range(_NSUB)]

    def _compute(b):
        def _grp(jg, c2):
            sl = pl.ds(jg * 16, 16)
            si = eb[b][0, sl]
            di = eb[b][1, sl]
            z = plsc.load_gather(ssrc_v, [si]) + plsc.load_gather(sdst_v, [di])
            l = jnp.maximum(z, 0.2 * z)
            p = jnp.exp(l - mvec)
            plsc.addupdate_scatter(den_v, [di], p)
            q = p * plsc.bitcast(eb[b][2, sl], jnp.float32)
            for e0 in range(16):
                qs = q[e0]
                er = jg * 16 + e0
                for c4 in range(_DH // 16):
                    cs = pl.ds(c4 * 16, 16)
                    rows[b][er, cs] = rows[b][er, cs] * qs
            return c2

        lax.fori_loop(0, _C // 16, _grp, 0)

    def _chunk(k, carry):
        pltpu.sync_copy(ed_h.at[wid, k], eb[0])
        _fill_d2(0)
        for cp in _start_gathers(0):
            cp.wait()
        _compute(0)
        for cp in _start_scatters(0):
            cp.wait()
        return carry

    lax.fori_loop(0, _CPW, _chunk, 0)
    plsc.subcore_barrier()

    pltpu.sync_copy(den_v.at[pl.ds(0, _N)], den_o.at[wid, 0])
    pltpu.sync_copy(out_sh.at[pl.ds(sid * _STRIPE, _STRIPE)],
                    raw_o.at[cid, pl.ds(sid * _STRIPE, _STRIPE)])


def _edge(edata, hp, ssrc, sdst, mvec):
    mesh = plsc.VectorSubcoreMesh(core_axis_name="c", subcore_axis_name="s")
    kern = pl.kernel(
        _edge_body,
        mesh=mesh,
        compiler_params=pltpu.CompilerParams(needs_layout_passes=False,
                                             use_tc_tiling_on_sc=False),
        out_type=[
            jax.ShapeDtypeStruct((2, _N, _DH), jnp.float32),
            jax.ShapeDtypeStruct((_NW, 1, _N), jnp.float32),
        ],
        scratch_types=[
            pltpu.VMEM((_N,), jnp.float32),
            pltpu.VMEM((_NPAD,), jnp.float32),
            pltpu.VMEM((_NPAD,), jnp.float32),
            pltpu.VMEM((3, _C), jnp.int32),
            pltpu.VMEM((_NSUB, _SUB), jnp.int32),
            pltpu.VMEM((_C, _DH), jnp.float32),
            pltpu.VMEM((16,), jnp.float32),
            pltpu.VMEM((16,), jnp.float32),
            pltpu.VMEM_SHARED((_NPAD, _DH), jnp.float32),
            pltpu.SemaphoreType.DMA,
            pltpu.SemaphoreType.DMA,
        ],
    )
    return kern(edata, hp, ssrc, sdst, mvec)


def kernel(x, edge_index_quote, edge_index_reply, edge_index_mention,
           edge_weight_quote, edge_weight_reply, edge_weight_mention, params):
    rels = ("quote", "reply", "mention")

    def pack(ei, ew):
        def padw(a, padval):
            a2 = a.reshape(_NW, _EPW)
            pad = jnp.full((_NW, _EPWP - _EPW), padval, a.dtype)
            return jnp.concatenate([a2, pad], 1).reshape(_NW, _CPW, _C)

        return jnp.stack(
            [padw(ei[0], 0), padw(ei[1], _N),
             padw(lax.bitcast_convert_type(ew, jnp.int32), 0)], axis=2)

    packed = (pack(edge_index_quote, edge_weight_quote),
              pack(edge_index_reply, edge_weight_reply),
              pack(edge_index_mention, edge_weight_mention))

    def edge_layer(hp, ssrc, sdst, m):
        mv = jnp.broadcast_to(m.reshape(()), (16,))
        raws, dens = [], []
        for r in range(3):
            raw_r, den_r = _edge(packed[r], hp[r],
                                 ssrc[r, 0], sdst[r, 0], mv)
            raws.append(raw_r)
            dens.append(den_r)
        return (jnp.stack(raws, axis=1).reshape(2, 3, _N, _DH),
                jnp.stack(dens, axis=1).reshape(_NW, 3, 1, _N))
    Ws = [jnp.stack([params[r][li]["W"] for r in rels]) for li in range(3)]
    Asrc = [jnp.stack([params[r][li]["a_src"] for r in rels]).reshape(3, 1, _DH)
            for li in range(3)]
    Adst = [jnp.stack([params[r][li]["a_dst"] for r in rels]).reshape(3, 1, _DH)
            for li in range(3)]
    Bs = [jnp.stack([params[r][li]["b"] for r in rels]).reshape(3, 1, _DH)
          for li in range(3)]

    hp, ssrc, sdst, m = _proj0(x, Ws[0], Asrc[0], Adst[0])
    for li in (1, 2):
        raw, den = edge_layer(hp, ssrc, sdst, m)
        hp, ssrc, sdst, m = _finproj(raw, den, Bs[li - 1],
                                     Ws[li], Asrc[li], Adst[li])
    raw, den = edge_layer(hp, ssrc, sdst, m)
    out3 = _finlast(raw, den, Bs[2])
    return jnp.concatenate([out3[0], out3[1], out3[2]], axis=1)


# speedup vs baseline: 2.1192x; 2.1192x over previous
import functools

import jax
import jax.numpy as jnp
from jax import lax
from jax.experimental import pallas as pl
from jax.experimental.pallas import tpu as pltpu
from jax.experimental.pallas import tpu_sc as plsc

_N = 10000
_DIN = 128
_DH = 64
_E = 320000
_N3 = 3 * _N
_E3 = 3 * _E
_NW = 32
_EPW = _E // _NW
_C = 400
_CPW = 25
_EPWP = _CPW * _C
_SUB = 80
_NSUB = _C // _SUB
_NTILE = 16
_STRIPE = _N // _NTILE


def _proj_tail(r, hp, asrc_ref, adst_ref, hp_ref, ssrc_ref, sdst_ref, m_ref):
    hp_ref[0] = hp
    ssrc = jnp.sum(hp * asrc_ref[0, 0][None, :], axis=1)
    sdst = jnp.sum(hp * adst_ref[0, 0][None, :], axis=1)
    ssrc_ref[0, 0] = ssrc
    sdst_ref[0, 0] = sdst
    z = jnp.max(ssrc) + jnp.max(sdst)
    lm = jnp.maximum(z, 0.2 * z)

    @pl.when(r == 0)
    def _():
        m_ref[0, 0] = lm

    @pl.when(r != 0)
    def _():
        m_ref[0, 0] = jnp.maximum(m_ref[0, 0], lm)


def _proj0_body(x_ref, w_ref, asrc_ref, adst_ref,
                hp_ref, ssrc_ref, sdst_ref, m_ref):
    r = pl.program_id(0)
    hp = jnp.dot(x_ref[...], w_ref[0], preferred_element_type=jnp.float32)
    _proj_tail(r, hp, asrc_ref, adst_ref, hp_ref, ssrc_ref, sdst_ref, m_ref)


def _finish(raw_ref, den_ref, b_ref):
    raw = raw_ref[0, 0] + raw_ref[1, 0]
    den = jnp.sum(den_ref[:, 0, 0, :], axis=0)
    return raw / (den + 1e-16)[:, None] + b_ref[0, 0][None, :]


def _finproj_body(raw_ref, den_ref, b_ref, w_ref, asrc_ref, adst_ref,
                  hp_ref, ssrc_ref, sdst_ref, m_ref):
    r = pl.program_id(0)
    h = jnp.maximum(_finish(raw_ref, den_ref, b_ref), 0.0)
    hp = jnp.dot(h, w_ref[0], preferred_element_type=jnp.float32)
    _proj_tail(r, hp, asrc_ref, adst_ref, hp_ref, ssrc_ref, sdst_ref, m_ref)


def _finlast_body(raw_ref, den_ref, b_ref, out_ref):
    out_ref[0] = _finish(raw_ref, den_ref, b_ref)


_PROJ_OUT_SHAPE = [
    jax.ShapeDtypeStruct((3, _N, _DH), jnp.float32),
    jax.ShapeDtypeStruct((3, 1, _N), jnp.float32),
    jax.ShapeDtypeStruct((3, 1, _N), jnp.float32),
    jax.ShapeDtypeStruct((1, 1), jnp.float32),
]
_PROJ_OUT_SPECS = [
    pl.BlockSpec((1, _N, _DH), lambda r: (r, 0, 0)),
    pl.BlockSpec((1, 1, _N), lambda r: (r, 0, 0)),
    pl.BlockSpec((1, 1, _N), lambda r: (r, 0, 0)),
    pl.BlockSpec(memory_space=pltpu.SMEM),
]
_W_SPEC0 = pl.BlockSpec((1, _DIN, _DH), lambda r: (r, 0, 0))
_W_SPEC = pl.BlockSpec((1, _DH, _DH), lambda r: (r, 0, 0))
_A_SPEC = pl.BlockSpec((1, 1, _DH), lambda r: (r, 0, 0))
_RAW_SPEC = pl.BlockSpec((2, 1, _N, _DH), lambda r: (0, r, 0, 0))
_DEN_SPEC = pl.BlockSpec((_NW, 1, 1, _N), lambda r: (0, r, 0, 0))


def _proj0(x, w, asrc, adst):
    return pl.pallas_call(
        _proj0_body,
        grid=(3,),
        in_specs=[
            pl.BlockSpec((_N, _DIN), lambda r: (0, 0)),
            _W_SPEC0, _A_SPEC, _A_SPEC,
        ],
        out_specs=_PROJ_OUT_SPECS,
        out_shape=_PROJ_OUT_SHAPE,
    )(x, w, asrc, adst)


def _finproj(raw, den, b, w, asrc, adst):
    return pl.pallas_call(
        _finproj_body,
        grid=(3,),
        in_specs=[_RAW_SPEC, _DEN_SPEC, _A_SPEC, _W_SPEC, _A_SPEC, _A_SPEC],
        out_specs=_PROJ_OUT_SPECS,
        out_shape=_PROJ_OUT_SHAPE,
    )(raw, den, b, w, asrc, adst)


def _finlast(raw, den, b):
    return pl.pallas_call(
        _finlast_body,
        grid=(3,),
        in_specs=[_RAW_SPEC, _DEN_SPEC, _A_SPEC],
        out_specs=pl.BlockSpec((1, _N, _DH), lambda r: (r, 0, 0)),
        out_shape=jax.ShapeDtypeStruct((3, _N, _DH), jnp.float32),
    )(raw, den, b)


_SENT = -1e30
_NPAD = _N + 16


def _edge_body(ed_h, hp_h, ssrc_h, sdst_h,
               m_h, raw_o, den_o,
               ssrc_v, sdst_v, den_v, eb0, d20,
               rows0, m_v, qbuf, out_sh, sg0, ss0):
    cid = lax.axis_index("c")
    sid = lax.axis_index("s")
    wid = cid * _NTILE + sid
    zeros16 = jnp.zeros((16,), jnp.float32)
    sent16 = jnp.full((16,), _SENT, jnp.float32)
    iota16 = lax.iota(jnp.int32, 16)
    eb = (eb0,)
    d2 = (d20,)
    rows = (rows0,)
    sg = (sg0,)
    ss = (ss0,)

    pltpu.sync_copy(ssrc_h, ssrc_v)
    pltpu.sync_copy(sdst_h, sdst_v.at[pl.ds(0, _N)])
    sdst_v[pl.ds(_N, 16)] = sent16
    pltpu.sync_copy(m_h, m_v)

    def _zden(i, carry):
        den_v[pl.ds(i * 16, 16)] = zeros16
        return carry

    lax.fori_loop(0, _NPAD // 16, _zden, 0)

    def _zrow(i, carry):
        for c4 in range(_DH // 16):
            rows0[i, pl.ds(c4 * 16, 16)] = zeros16
        return carry

    lax.fori_loop(0, _C, _zrow, 0)
    zst = _NPAD // _NTILE
    pltpu.sync_copy(rows0.at[pl.ds(0, _C)],
                    out_sh.at[pl.ds(sid * zst, _C)])
    pltpu.sync_copy(rows0.at[pl.ds(0, zst - _C)],
                    out_sh.at[pl.ds(sid * zst + _C, zst - _C)])
    plsc.subcore_barrier()

    mvec = m_v[...]

    def _fill_d2(b):
        for j in range(_NSUB):
            for g in range(_SUB // 16):
                d2[b][j, pl.ds(g * 16, 16)] = \
                    eb[b][1, pl.ds(j * _SUB + g * 16, 16)]

    def _start_gathers(b):
        return [pltpu.async_copy(
                    hp_h.at[eb[b].at[0, pl.ds(j * _SUB, _SUB)]],
                    rows[b].at[pl.ds(j * _SUB, _SUB)], sg[b])
                for j in range(_NSUB)]

    def _start_scatters(b):
        return [pltpu.async_copy(rows[b].at[pl.ds(j * _SUB, _SUB)],
                                 out_sh.at[d2[b].at[j]], ss[b], add=True)
                for j in range(_NSUB)]

    def _compute(b):
        def _grp(jg, c2):
            sl = pl.ds(jg * 16, 16)
            si = eb[b][0, sl]
            di = eb[b][1, sl]
            z = plsc.load_gather(ssrc_v, [si]) + plsc.load_gather(sdst_v, [di])
            l = jnp.maximum(z, 0.2 * z)
            p = jnp.exp(l - mvec)
            plsc.addupdate_scatter(den_v, [di], p)
            q = p * plsc.bitcast(eb[b][2, sl], jnp.float32)
            for e0 in range(16):
                qs = q[e0]
                er = jg * 16 + e0
                for c4 in range(_DH // 16):
                    cs = pl.ds(c4 * 16, 16)
                    rows[b][er, cs] = rows[b][er, cs] * qs
            return c2

        lax.fori_loop(0, _C // 16, _grp, 0)

    def _chunk(k, carry):
        pltpu.sync_copy(ed_h.at[wid, k], eb[0])
        _fill_d2(0)
        for cp in _start_gathers(0):
            cp.wait()
        _compute(0)
        for cp in _start_scatters(0):
            cp.wait()
        return carry

    lax.fori_loop(0, _CPW, _chunk, 0)
    plsc.subcore_barrier()

    pltpu.sync_copy(den_v.at[pl.ds(0, _N)], den_o.at[wid, 0])
    pltpu.sync_copy(out_sh.at[pl.ds(sid * _STRIPE, _STRIPE)],
                    raw_o.at[cid, pl.ds(sid * _STRIPE, _STRIPE)])


def _edge(edata, hp, ssrc, sdst, mvec):
    mesh = plsc.VectorSubcoreMesh(core_axis_name="c", subcore_axis_name="s")
    kern = pl.kernel(
        _edge_body,
        mesh=mesh,
        compiler_params=pltpu.CompilerParams(needs_layout_passes=False,
                                             use_tc_tiling_on_sc=False),
        out_type=[
            jax.ShapeDtypeStruct((2, _N, _DH), jnp.float32),
            jax.ShapeDtypeStruct((_NW, 1, _N), jnp.float32),
        ],
        scratch_types=[
            pltpu.VMEM((_N,), jnp.float32),
            pltpu.VMEM((_NPAD,), jnp.float32),
            pltpu.VMEM((_NPAD,), jnp.float32),
            pltpu.VMEM((3, _C), jnp.int32),
            pltpu.VMEM((_NSUB, _SUB), jnp.int32),
            pltpu.VMEM((_C, _DH), jnp.float32),
            pltpu.VMEM((16,), jnp.float32),
            pltpu.VMEM((16,), jnp.float32),
            pltpu.VMEM_SHARED((_NPAD, _DH), jnp.float32),
            pltpu.SemaphoreType.DMA,
            pltpu.SemaphoreType.DMA,
        ],
    )
    return kern(edata, hp, ssrc, sdst, mvec)


def kernel(x, edge_index_quote, edge_index_reply, edge_index_mention,
           edge_weight_quote, edge_weight_reply, edge_weight_mention, params):
    rels = ("quote", "reply", "mention")

    def pack(ei, ew):
        def padw(a, padval):
            a2 = a.reshape(_NW, _EPW)
            if _EPWP > _EPW:
                pad = jnp.full((_NW, _EPWP - _EPW), padval, a.dtype)
                a2 = jnp.concatenate([a2, pad], 1)
            return a2.reshape(_NW, _CPW, _C)

        return jnp.stack(
            [padw(ei[0], 0), padw(ei[1], _N),
             padw(lax.bitcast_convert_type(ew, jnp.int32), 0)], axis=2)

    packed = (pack(edge_index_quote, edge_weight_quote),
              pack(edge_index_reply, edge_weight_reply),
              pack(edge_index_mention, edge_weight_mention))

    def edge_layer(hp, ssrc, sdst, m):
        mv = jnp.broadcast_to(m.reshape(()), (16,))
        raws, dens = [], []
        for r in range(3):
            raw_r, den_r = _edge(packed[r], hp[r],
                                 ssrc[r, 0], sdst[r, 0], mv)
            raws.append(raw_r)
            dens.append(den_r)
        return (jnp.stack(raws, axis=1).reshape(2, 3, _N, _DH),
                jnp.stack(dens, axis=1).reshape(_NW, 3, 1, _N))
    Ws = [jnp.stack([params[r][li]["W"] for r in rels]) for li in range(3)]
    Asrc = [jnp.stack([params[r][li]["a_src"] for r in rels]).reshape(3, 1, _DH)
            for li in range(3)]
    Adst = [jnp.stack([params[r][li]["a_dst"] for r in rels]).reshape(3, 1, _DH)
            for li in range(3)]
    Bs = [jnp.stack([params[r][li]["b"] for r in rels]).reshape(3, 1, _DH)
          for li in range(3)]

    hp, ssrc, sdst, m = _proj0(x, Ws[0], Asrc[0], Adst[0])
    for li in (1, 2):
        raw, den = edge_layer(hp, ssrc, sdst, m)
        hp, ssrc, sdst, m = _finproj(raw, den, Bs[li - 1],
                                     Ws[li], Asrc[li], Adst[li])
    raw, den = edge_layer(hp, ssrc, sdst, m)
    out3 = _finlast(raw, den, Bs[2])
    return jnp.concatenate([out3[0], out3[1], out3[2]], axis=1)
